# baseline (speedup 1.0000x reference)
import functools

import ml_dtypes
import numpy as np
import jax
import jax.numpy as jnp
from jax import lax
from jax.experimental import pallas as pl
from jax.experimental.pallas import tpu as pltpu
from jax.experimental.pallas import tpu_sc as plsc

_SEQ = 2048
_HID = 1024
_BATCH = 4
_ROPE_BASE = 10000.0

_NW = 32
_POS_W = _SEQ // _NW
_PCH = 16
_NCH = _POS_W // _PCH
_NBLK = _NCH * _BATCH
_UNIT = _HID // 32
_NBUF = 3


def _rope_table_packed():
    i = np.arange(0, _HID, 2, dtype=np.float64)
    theta = _ROPE_BASE ** (-2.0 * i / _HID)
    m = np.arange(_SEQ, dtype=np.float64)[:, None] * theta[None, :]
    packed = np.empty((_SEQ, _HID), dtype=np.float32)
    packed[:, 0::2] = np.cos(m)
    packed[:, 1::2] = np.sin(m)
    return packed.reshape(-1)


_PACKED_NP = _rope_table_packed()

_mesh = plsc.VectorSubcoreMesh(core_axis_name="c", subcore_axis_name="s")


@functools.partial(
    pl.kernel,
    out_type=jax.ShapeDtypeStruct((_BATCH * _SEQ, _HID), jnp.float32),
    mesh=_mesh,
    compiler_params=pltpu.CompilerParams(needs_layout_passes=False),
    scratch_types=[
        pltpu.VMEM((_POS_W * _HID,), jnp.float32),
        pltpu.VMEM((_BATCH, _POS_W), jnp.int32),
        pltpu.VMEM((_NBUF, _PCH, _HID), jnp.float32),
        pltpu.SemaphoreType.DMA,
        pltpu.SemaphoreType.DMA,
        pltpu.SemaphoreType.DMA,
        pltpu.SemaphoreType.DMA,
        pltpu.SemaphoreType.DMA,
        pltpu.SemaphoreType.DMA,
        pltpu.SemaphoreType.DMA,
        pltpu.SemaphoreType.DMA,
    ],
)
def _rope_sc(x_hbm, tab_hbm, p_hbm, out_hbm,
             p_v, idx_v, rows_v,
             psem, isem, gsem0, gsem1, gsem2, ssem0, ssem1, ssem2):
    wid = lax.axis_index("s") * 2 + lax.axis_index("c")
    iota = lax.iota(jnp.int32, 16)
    ev_idx = iota * 2
    od_idx = ev_idx + 1

    gsem_b = (gsem0, gsem1, gsem2)
    ssem_b = (ssem0, ssem1, ssem2)

    pos0 = wid * _POS_W
    pcopy = pltpu.async_copy(p_hbm.at[pl.ds(pos0 * _HID, _POS_W * _HID)],
                             p_v, psem)
    icopies = [
        pltpu.async_copy(x_hbm.at[pl.ds(b * _SEQ + pos0, _POS_W)],
                         idx_v.at[b], isem)
        for b in range(_BATCH)
    ]

    def blk_base(k):
        c, b = divmod(k, _BATCH)
        return b * _SEQ + pos0 + c * _PCH

    def start_gather(k):
        c, b = divmod(k, _BATCH)
        return pltpu.async_copy(
            tab_hbm.at[idx_v.at[b, pl.ds(c * _PCH, _PCH)]],
            rows_v.at[k % _NBUF], gsem_b[k % _NBUF])

    for ic in icopies:
        ic.wait()
    gh = {0: start_gather(0), 1: start_gather(1)}
    sh = {}
    pcopy.wait()
    for k in range(_NBLK):
        if k + 2 < _NBLK:
            if k - 1 in sh:
                sh.pop(k - 1).wait()
            gh[k + 2] = start_gather(k + 2)
        gh.pop(k).wait()

        buf = rows_v.at[k % _NBUF]
        c_blk = k // _BATCH

        @plsc.parallel_loop(0, _PCH * _UNIT, unroll=4)
        def _(i):
            r = i // _UNIT
            h = i - r * _UNIT
            rvec = jnp.full((16,), r, dtype=jnp.int32)
            ce = ev_idx + h * 32
            co = od_idx + h * 32
            ev = plsc.load_gather(buf, [rvec, ce])
            od = plsc.load_gather(buf, [rvec, co])
            poff = ((c_blk * _PCH + r) * _UNIT + h) * 32
            cv = plsc.load_gather(p_v, [poff + ev_idx])
            sv = plsc.load_gather(p_v, [poff + od_idx])
            plsc.store_scatter(buf, [rvec, ce], ev * cv - od * sv)
            plsc.store_scatter(buf, [rvec, co], od * cv + ev * sv)

        sh[k] = pltpu.async_copy(buf, out_hbm.at[pl.ds(blk_base(k), _PCH)],
                                 ssem_b[k % _NBUF])
    sh.pop(_NBLK - 2).wait()
    sh.pop(_NBLK - 1).wait()


def kernel(x, table):
    out = _rope_sc(x.reshape(-1), table, jnp.asarray(_PACKED_NP))
    return out.reshape(_BATCH, _SEQ, _HID)


# speedup vs baseline: 1.0373x; 1.0373x over previous
import functools

import ml_dtypes
import numpy as np
import jax
import jax.numpy as jnp
from jax import lax
from jax.experimental import pallas as pl
from jax.experimental.pallas import tpu as pltpu
from jax.experimental.pallas import tpu_sc as plsc

_SEQ = 2048
_HID = 1024
_BATCH = 4
_ROPE_BASE = 10000.0

_NW = 32
_POS_W = _SEQ // _NW
_PCH = 16
_NCH = _POS_W // _PCH
_NBLK = _NCH * _BATCH
_UNIT = _HID // 32
_NBUF = 3


def _rope_table_packed():
    i = np.arange(0, _HID, 2, dtype=np.float64)
    theta = _ROPE_BASE ** (-2.0 * i / _HID)
    m = np.arange(_SEQ, dtype=np.float64)[:, None] * theta[None, :]
    cos = np.cos(m).astype(ml_dtypes.bfloat16).view(np.uint16)
    sin = np.sin(m).astype(ml_dtypes.bfloat16).view(np.uint16)
    words = cos.astype(np.uint32) | (sin.astype(np.uint32) << 16)
    return words.view(np.int32).reshape(-1)


_PACKED_NP = _rope_table_packed()

_mesh = plsc.VectorSubcoreMesh(core_axis_name="c", subcore_axis_name="s")


@functools.partial(
    pl.kernel,
    out_type=jax.ShapeDtypeStruct((_BATCH * _SEQ, _HID), jnp.float32),
    mesh=_mesh,
    compiler_params=pltpu.CompilerParams(needs_layout_passes=False),
    scratch_types=[
        pltpu.VMEM((_POS_W * _HID // 2,), jnp.int32),
        pltpu.VMEM((_BATCH, _POS_W), jnp.int32),
        pltpu.VMEM((_NBUF, _PCH, _HID), jnp.float32),
        pltpu.SemaphoreType.DMA,
        pltpu.SemaphoreType.DMA,
        pltpu.SemaphoreType.DMA,
        pltpu.SemaphoreType.DMA,
        pltpu.SemaphoreType.DMA,
        pltpu.SemaphoreType.DMA,
        pltpu.SemaphoreType.DMA,
        pltpu.SemaphoreType.DMA,
    ],
)
def _rope_sc(x_hbm, tab_hbm, p_hbm, out_hbm,
             p_v, idx_v, rows_v,
             psem, isem, gsem0, gsem1, gsem2, ssem0, ssem1, ssem2):
    wid = lax.axis_index("s") * 2 + lax.axis_index("c")
    iota = lax.iota(jnp.int32, 16)
    ev_idx = iota * 2
    od_idx = ev_idx + 1

    gsem_b = (gsem0, gsem1, gsem2)
    ssem_b = (ssem0, ssem1, ssem2)

    pos0 = wid * _POS_W
    pcopy = pltpu.async_copy(
        p_hbm.at[pl.ds(pos0 * (_HID // 2), _POS_W * _HID // 2)], p_v, psem)
    icopies = [
        pltpu.async_copy(x_hbm.at[pl.ds(b * _SEQ + pos0, _POS_W)],
                         idx_v.at[b], isem)
        for b in range(_BATCH)
    ]

    def blk_base(k):
        c, b = divmod(k, _BATCH)
        return b * _SEQ + pos0 + c * _PCH

    def start_gather(k):
        c, b = divmod(k, _BATCH)
        return pltpu.async_copy(
            tab_hbm.at[idx_v.at[b, pl.ds(c * _PCH, _PCH)]],
            rows_v.at[k % _NBUF], gsem_b[k % _NBUF])

    for ic in icopies:
        ic.wait()
    gh = {0: start_gather(0), 1: start_gather(1)}
    sh = {}
    pcopy.wait()
    for k in range(_NBLK):
        if k + 2 < _NBLK:
            if k - 1 in sh:
                sh.pop(k - 1).wait()
            gh[k + 2] = start_gather(k + 2)
        gh.pop(k).wait()

        buf = rows_v.at[k % _NBUF]
        c_blk = k // _BATCH

        @plsc.parallel_loop(0, _PCH * _UNIT, unroll=4)
        def _(i):
            r = i // _UNIT
            h = i - r * _UNIT
            rvec = jnp.full((16,), r, dtype=jnp.int32)
            ce = ev_idx + h * 32
            co = od_idx + h * 32
            ev = plsc.load_gather(buf, [rvec, ce])
            od = plsc.load_gather(buf, [rvec, co])
            woff = ((c_blk * _PCH + r) * _UNIT + h) * 16
            w = plsc.load_gather(p_v, [woff + iota])
            cv, sv = plsc.unpack(plsc.bitcast(w, jnp.bfloat16),
                                 format=plsc.PackFormat.INTERLEAVED)
            plsc.store_scatter(buf, [rvec, ce], ev * cv - od * sv)
            plsc.store_scatter(buf, [rvec, co], od * cv + ev * sv)

        sh[k] = pltpu.async_copy(buf, out_hbm.at[pl.ds(blk_base(k), _PCH)],
                                 ssem_b[k % _NBUF])
    sh.pop(_NBLK - 2).wait()
    sh.pop(_NBLK - 1).wait()


def kernel(x, table):
    out = _rope_sc(x.reshape(-1), table, jnp.asarray(_PACKED_NP))
    return out.reshape(_BATCH, _SEQ, _HID)
